# Initial kernel scaffold; baseline (speedup 1.0000x reference)
import jax
import jax.numpy as jnp
from jax.experimental import pallas as pl


def kernel(gt, pred):
    raise NotImplementedError("write your pallas kernel here")


# speedup vs baseline: 1.4917x; 1.4917x over previous
import jax
import jax.numpy as jnp
from jax.experimental import pallas as pl
from jax.experimental.pallas import tpu as pltpu

_EPS = 1e-8
_IGNORE = 255


def _iou_body(n_classes, n_chunks, gt_ref, pred_ref, out_ref):
    c = pl.program_id(1)
    g = gt_ref[...]
    p = pred_ref[...]
    eq = g == p
    valid = g != _IGNORE
    gt_in = (g >= 1) & (g < n_classes)
    pred_in = (p >= 1) & (p < n_classes)
    tp = jnp.sum((eq & gt_in & valid).astype(jnp.float32))
    fp = jnp.sum((~eq & pred_in & valid).astype(jnp.float32))
    fn = jnp.sum((~eq & gt_in & valid).astype(jnp.float32))
    vals = jnp.broadcast_to(
        jnp.stack([tp, fp, fn]).reshape(1, 3, 1), (1, 3, 128)
    )

    @pl.when(c == 0)
    def _():
        out_ref[...] = vals

    @pl.when(c != 0)
    def _():
        out_ref[...] += vals


def kernel(gt, pred, interpret=False):
    B, C, H, W = gt.shape
    n_classes = pred.shape[1]
    CBLK = 3
    n_chunks = (C + CBLK - 1) // CBLK
    pad = n_chunks * CBLK - C
    if pad:
        gt = jnp.pad(gt, ((0, 0), (0, pad), (0, 0), (0, 0)),
                     constant_values=_IGNORE)
        pred = jnp.pad(pred, ((0, 0), (0, pad), (0, 0), (0, 0)),
                       constant_values=0)

    import functools
    body = functools.partial(_iou_body, n_classes, n_chunks)
    out = pl.pallas_call(
        body,
        out_shape=jax.ShapeDtypeStruct((B, 3, 128), jnp.float32),
        grid=(B, n_chunks),
        in_specs=[
            pl.BlockSpec((1, CBLK, H, W), lambda b, c: (b, c, 0, 0)),
            pl.BlockSpec((1, CBLK, H, W), lambda b, c: (b, c, 0, 0)),
        ],
        out_specs=pl.BlockSpec((1, 3, 128), lambda b, c: (b, 0, 0)),
        compiler_params=pltpu.CompilerParams(
            dimension_semantics=("parallel", "arbitrary"),
            vmem_limit_bytes=56 * 1024 * 1024,
        ),
        name="iou_counts",
        interpret=interpret,
    )(gt, pred)

    tp = out[:, 0, 0]
    fp = out[:, 1, 0]
    fn = out[:, 2, 0]
    jac = tp / jnp.maximum(tp + fp + fn, _EPS)
    return jnp.mean(jac)


# speedup vs baseline: 2.3793x; 1.5951x over previous
import functools

import jax
import jax.numpy as jnp
from jax.experimental import pallas as pl
from jax.experimental.pallas import tpu as pltpu

_EPS = 1e-8
_IGNORE = 255


def _iou_body(n_classes, gt_ref, pred_ref, out_ref):
    c = pl.program_id(1)
    g = gt_ref[...]
    p = pred_ref[...]
    nm1 = jnp.uint32(n_classes - 1)
    a = (g - 1).astype(jnp.uint32) < nm1
    b = ((p - 1).astype(jnp.uint32) < nm1) & (g != _IGNORE)
    t = a & (g == p)
    a_s = jnp.sum(a.astype(jnp.float32))
    b_s = jnp.sum(b.astype(jnp.float32))
    t_s = jnp.sum(t.astype(jnp.float32))
    vals = jnp.broadcast_to(
        jnp.stack([t_s, a_s, b_s]).reshape(1, 3, 1), (1, 3, 128)
    )

    @pl.when(c == 0)
    def _():
        out_ref[...] = vals

    @pl.when(c != 0)
    def _():
        out_ref[...] += vals


def kernel(gt, pred, interpret=False):
    B, C, H, W = gt.shape
    n_classes = pred.shape[1]
    CBLK = 3
    n_chunks = (C + CBLK - 1) // CBLK
    pad = n_chunks * CBLK - C
    if pad:
        gt = jnp.pad(gt, ((0, 0), (0, pad), (0, 0), (0, 0)),
                     constant_values=_IGNORE)
        pred = jnp.pad(pred, ((0, 0), (0, pad), (0, 0), (0, 0)),
                       constant_values=0)

    body = functools.partial(_iou_body, n_classes)
    out = pl.pallas_call(
        body,
        out_shape=jax.ShapeDtypeStruct((B, 3, 128), jnp.float32),
        grid=(B, n_chunks),
        in_specs=[
            pl.BlockSpec((1, CBLK, H, W), lambda b, c: (b, c, 0, 0)),
            pl.BlockSpec((1, CBLK, H, W), lambda b, c: (b, c, 0, 0)),
        ],
        out_specs=pl.BlockSpec((1, 3, 128), lambda b, c: (b, 0, 0)),
        compiler_params=pltpu.CompilerParams(
            dimension_semantics=("parallel", "arbitrary"),
            vmem_limit_bytes=56 * 1024 * 1024,
        ),
        name="iou_counts",
        interpret=interpret,
    )(gt, pred)

    tp = out[:, 0, 0]
    a = out[:, 1, 0]
    b = out[:, 2, 0]
    jac = tp / jnp.maximum(a + b - tp, _EPS)
    return jnp.mean(jac)


# speedup vs baseline: 2.8349x; 1.1914x over previous
import functools

import jax
import jax.numpy as jnp
from jax.experimental import pallas as pl
from jax.experimental.pallas import tpu as pltpu

_EPS = 1e-8
_IGNORE = 255


def _iou_body(n_classes, gt_ref, pred_ref, out_ref):
    c = pl.program_id(1)
    g = gt_ref[...]
    p = pred_ref[...]
    nm1 = jnp.uint32(n_classes - 1)
    a = (g - 1).astype(jnp.uint32) < nm1
    b = ((p - 1).astype(jnp.uint32) < nm1) & (g != _IGNORE)
    t = a & (g == p)
    a_s = jnp.sum(a.astype(jnp.float32))
    b_s = jnp.sum(b.astype(jnp.float32))
    t_s = jnp.sum(t.astype(jnp.float32))
    vals = jnp.broadcast_to(
        jnp.stack([t_s, a_s, b_s]).reshape(1, 3, 1), (1, 3, 128)
    )

    @pl.when(c == 0)
    def _():
        out_ref[...] = vals

    @pl.when(c != 0)
    def _():
        out_ref[...] += vals


def kernel(gt, pred, interpret=False):
    B, C, H, W = gt.shape
    n_classes = pred.shape[1]
    R = C * H
    n_chunks = 2
    while (R % n_chunks) or (R // n_chunks) * W * 4 > 11 * 1024 * 1024:
        n_chunks += 1
    ROWS = R // n_chunks
    gt = gt.reshape(B, R, W)
    pred = pred.reshape(B, R, W)

    body = functools.partial(_iou_body, n_classes)
    out = pl.pallas_call(
        body,
        out_shape=jax.ShapeDtypeStruct((B, 3, 128), jnp.float32),
        grid=(B, n_chunks),
        in_specs=[
            pl.BlockSpec((1, ROWS, W), lambda b, c: (b, c, 0)),
            pl.BlockSpec((1, ROWS, W), lambda b, c: (b, c, 0)),
        ],
        out_specs=pl.BlockSpec((1, 3, 128), lambda b, c: (b, 0, 0)),
        compiler_params=pltpu.CompilerParams(
            dimension_semantics=("parallel", "arbitrary"),
            vmem_limit_bytes=56 * 1024 * 1024,
        ),
        name="iou_counts",
        interpret=interpret,
    )(gt, pred)

    tp = out[:, 0, 0]
    a = out[:, 1, 0]
    b = out[:, 2, 0]
    jac = tp / jnp.maximum(a + b - tp, _EPS)
    return jnp.mean(jac)
